# baseline (speedup 1.0000x reference)
import jax
import jax.numpy as jnp
from jax import lax
from jax.experimental import pallas as pl
from jax.experimental.pallas import tpu as pltpu
from jax.experimental.pallas import tpu_sc as plsc

D = 128
B = 4096
K = 32
H = 2
D4 = 32
N_ITER = 30

_SELU_ALPHA = 1.6732632423543772
_SELU_SCALE = 1.0507009873554805

_NC, _NS = 2, 16
_NW = _NC * _NS
_NIDX = B * K + B
_RPW = _NIDX // _NW
_CH = 352
_NCHUNK = _RPW // _CH


def _sc_gather_body(table, idx_hbm, out_hbm, idx_v, buf, sem):
    wid = lax.axis_index("s") * _NC + lax.axis_index("c")
    base = wid * _RPW
    pltpu.sync_copy(idx_hbm.at[pl.ds(base, _RPW)], idx_v)
    for j in range(_NCHUNK):
        pltpu.async_copy(table.at[idx_v.at[pl.ds(j * _CH, _CH)]], buf, sem).wait()
        pltpu.sync_copy(buf, out_hbm.at[pl.ds(base + j * _CH, _CH)])


def _sc_gather(u2e, idx):
    f = pl.kernel(
        _sc_gather_body,
        mesh=plsc.VectorSubcoreMesh(core_axis_name="c", subcore_axis_name="s"),
        out_type=jax.ShapeDtypeStruct((_NIDX, D), jnp.float32),
        scratch_types=[
            pltpu.VMEM((_RPW,), jnp.int32),
            pltpu.VMEM((_CH, D), jnp.float32),
            pltpu.SemaphoreType.DMA,
        ],
    )
    return f(u2e, idx)


_TB = 256


def _normalize_rows(x):
    n = jnp.sqrt(jnp.sum(x * x, axis=-1, keepdims=True))
    return x / jnp.maximum(n, 1e-12)


def _selu(x):
    return _SELU_SCALE * jnp.where(x > 0, x, _SELU_ALPHA * (jnp.exp(x) - 1.0))


def _safe_pow(t, inv):
    pos = t > 0.0
    lg = jnp.log(jnp.where(pos, t, 1.0))
    return jnp.where(pos, jnp.exp(inv * lg), 0.0)


def _entmax_axis0(x, alpha):
    ones_row = jnp.ones((1, K), jnp.float32)
    am1 = alpha - 1.0
    xs = x * am1
    inv = 1.0 / am1
    mx = jnp.max(xs, axis=0, keepdims=True)
    lo = mx - 1.0
    hi = mx
    for _ in range(N_ITER):
        mid = 0.5 * (lo + hi)
        q = _safe_pow(jnp.maximum(xs - mid, 0.0), inv)
        f = jnp.dot(ones_row, q, preferred_element_type=jnp.float32,
                    precision=jax.lax.Precision.HIGHEST) - 1.0
        ge = f >= 0.0
        lo = jnp.where(ge, mid, lo)
        hi = jnp.where(ge, hi, mid)
    tau = 0.5 * (lo + hi)
    p = _safe_pow(jnp.maximum(xs - tau, 0.0), inv)
    return p / jnp.dot(ones_row, p, preferred_element_type=jnp.float32,
                       precision=jax.lax.Precision.HIGHEST)


def _attn_body(e_ref, s_ref, w1_ref, b1_ref, w2_ref, b2_ref, w3t_ref, b3_ref,
               l1t_ref, l1b_ref, out_ref):
    en = _normalize_rows(e_ref[...].reshape(K * _TB, D))
    en3 = en.reshape(K, _TB, D)
    u = s_ref[...]
    acc = jnp.zeros((_TB, D), jnp.float32)
    for h in range(H):
        u_n = _normalize_rows(u)
        a_e = jnp.dot(en, w1_ref[h, :D, :], preferred_element_type=jnp.float32)
        a_u = jnp.dot(u_n, w1_ref[h, D:, :], preferred_element_type=jnp.float32)
        b1 = b1_ref[h:h + 1, :].reshape(1, 1, D)
        a1 = _selu(a_e.reshape(K, _TB, D) + a_u[None, :, :] + b1)
        a2 = _selu(jnp.dot(a1.reshape(K * _TB, D), w2_ref[h],
                           preferred_element_type=jnp.float32)
                   + b2_ref[h:h + 1, :])
        w3 = w3t_ref[h].reshape(1, 1, D4)
        sT = jnp.sum(a2.reshape(K, _TB, D4) * w3, axis=-1) \
            + b3_ref[h:h + 1, 0:1]
        l1 = l1t_ref[h].reshape(1, 1, D)
        wT = jax.nn.sigmoid(jnp.sum(en3 * l1, axis=-1)
                            + l1b_ref[h:h + 1, 0:1]) + 1.0
        attT = _entmax_axis0(sT, wT)
        att = jnp.transpose(attT)
        u = jnp.zeros((_TB, D), jnp.float32)
        for k in range(K):
            u = u + en3[k] * att[:, k:k + 1]
        acc = acc + u
    out_ref[...] = acc * (1.0 / H)


def _full_spec(shape):
    return pl.BlockSpec(shape, lambda i: tuple(0 for _ in shape))


def _attn(e_kmaj, self_feats, att1_W, att1_b, att2_W, att2_b, att3_Wt, att3_b,
          lin1_Wt, lin1_b):
    return pl.pallas_call(
        _attn_body,
        grid=(B // _TB,),
        in_specs=[
            pl.BlockSpec((K, _TB, D), lambda i: (0, i, 0)),
            pl.BlockSpec((_TB, D), lambda i: (i, 0)),
            _full_spec((H, 2 * D, D)),
            _full_spec((H, D)),
            _full_spec((H, D, D4)),
            _full_spec((H, D4)),
            _full_spec((H, 1, D4)),
            _full_spec((H, 1)),
            _full_spec((H, 1, D)),
            _full_spec((H, 1)),
        ],
        out_specs=pl.BlockSpec((_TB, D), lambda i: (i, 0)),
        out_shape=jax.ShapeDtypeStruct((B, D), jnp.float32),
    )(e_kmaj, self_feats, att1_W, att1_b, att2_W, att2_b, att3_Wt, att3_b,
      lin1_Wt, lin1_b)


def _head_body(nf_ref, sf_ref, inw_ref, inb_ref, outw_ref, outb_ref, gw_ref,
               gb_ref, bng_ref, bnb_ref, bn1g_ref, bn1b_ref, o_ref):
    nf = nf_ref[...]
    sf = sf_ref[...]
    mu = jnp.mean(nf, axis=0, keepdims=True)
    xc = nf - mu
    var = jnp.mean(xc * xc, axis=0, keepdims=True)
    h0 = xc / jnp.sqrt(var + 1e-5) * bng_ref[...] + bnb_ref[...]
    h1 = _selu(jnp.dot(h0, inw_ref[...], preferred_element_type=jnp.float32)
               + inb_ref[...])
    mu1 = jnp.mean(h1, axis=0, keepdims=True)
    xc1 = h1 - mu1
    var1 = jnp.mean(xc1 * xc1, axis=0, keepdims=True)
    h1n = xc1 / jnp.sqrt(var1 + 1e-5) * bn1g_ref[...] + bn1b_ref[...]
    no = jnp.dot(h1n, outw_ref[...], preferred_element_type=jnp.float32) \
        + outb_ref[...]
    z = (jnp.dot(sf, gw_ref[0:D, :], preferred_element_type=jnp.float32)
         + jnp.dot(no, gw_ref[D:2 * D, :], preferred_element_type=jnp.float32)
         + jnp.dot(sf * no, gw_ref[2 * D:, :],
                   preferred_element_type=jnp.float32)
         + gb_ref[...])
    gama = jax.nn.sigmoid(z)
    o_ref[...] = gama * sf + (1.0 - gama) * no


def _head(nf, sf, in_W, in_b, out_W, out_b, gate_W, gate_b, bn_g, bn_b,
          bn1_g, bn1_b):
    return pl.pallas_call(
        _head_body,
        out_shape=jax.ShapeDtypeStruct((B, D), jnp.float32),
    )(nf, sf, in_W, in_b, out_W, out_b, gate_W, gate_b, bn_g, bn_b,
      bn1_g, bn1_b)


def kernel(nodes, neighbors, u2e, att1_W, att1_b, att2_W, att2_b, att3_W,
           att3_b, lin1_W, lin1_b, gate_W, gate_b, bn_g, bn_b, in_W, in_b,
           bn1_g, bn1_b, out_W, out_b):
    idx = jnp.concatenate([neighbors.T.reshape(-1).astype(jnp.int32),
                           nodes.astype(jnp.int32)])
    g = _sc_gather(u2e, idx)
    e_kmaj = g[:B * K].reshape(K, B, D)
    sf = g[B * K:]
    att3_Wt = jnp.transpose(att3_W, (0, 2, 1))
    lin1_Wt = jnp.transpose(lin1_W, (0, 2, 1))
    nf = _attn(e_kmaj, sf, att1_W, att1_b, att2_W, att2_b, att3_Wt, att3_b,
               lin1_Wt, lin1_b)
    return _head(nf, sf, in_W, in_b.reshape(1, D), out_W, out_b.reshape(1, D),
                 gate_W, gate_b.reshape(1, D), bn_g.reshape(1, D),
                 bn_b.reshape(1, D), bn1_g.reshape(1, D), bn1_b.reshape(1, D))


# speedup vs baseline: 6.8871x; 6.8871x over previous
import jax
import jax.numpy as jnp
from jax import lax
from jax.experimental import pallas as pl
from jax.experimental.pallas import tpu as pltpu
from jax.experimental.pallas import tpu_sc as plsc

D = 128
B = 4096
K = 32
H = 2
D4 = 32
N_ITER = 30

_SELU_ALPHA = 1.6732632423543772
_SELU_SCALE = 1.0507009873554805

_NC, _NS = 2, 16
_NW = _NC * _NS
_NIDX = B * K + B
_RPW = _NIDX // _NW
_CH = 352
_NCHUNK = _RPW // _CH


def _sc_gather_body(table, idx_hbm, out_hbm, idx_v, buf, sem):
    wid = lax.axis_index("s") * _NC + lax.axis_index("c")
    base = wid * _RPW
    pltpu.sync_copy(idx_hbm.at[pl.ds(base, _RPW)], idx_v)
    for j in range(_NCHUNK):
        pltpu.async_copy(table.at[idx_v.at[pl.ds(j * _CH, _CH)]], buf, sem).wait()
        pltpu.sync_copy(buf, out_hbm.at[pl.ds(base + j * _CH, _CH)])


def _sc_gather(u2e, idx):
    f = pl.kernel(
        _sc_gather_body,
        mesh=plsc.VectorSubcoreMesh(core_axis_name="c", subcore_axis_name="s"),
        out_type=jax.ShapeDtypeStruct((_NIDX, D), jnp.float32),
        scratch_types=[
            pltpu.VMEM((_RPW,), jnp.int32),
            pltpu.VMEM((_CH, D), jnp.float32),
            pltpu.SemaphoreType.DMA,
        ],
    )
    return f(u2e, idx)


_TB = 256


def _normalize_rows(x):
    n = jnp.sqrt(jnp.sum(x * x, axis=-1, keepdims=True))
    return x / jnp.maximum(n, 1e-12)


def _selu(x):
    return _SELU_SCALE * jnp.where(x > 0, x, _SELU_ALPHA * (jnp.exp(x) - 1.0))


def _safe_pow(t, inv):
    pos = t > 0.0
    lg = jnp.log(jnp.where(pos, t, 1.0))
    return jnp.where(pos, jnp.exp(inv * lg), 0.0)


def _full_spec(shape):
    return pl.BlockSpec(shape, lambda i: tuple(0 for _ in shape))


def _hop_scores(en, en3, u, h, w1_ref, b1_ref, w2_ref, b2_ref, w3t_ref,
                b3_ref):
    u_n = _normalize_rows(u)
    a_e = jnp.dot(en, w1_ref[h, :D, :], preferred_element_type=jnp.float32)
    a_u = jnp.dot(u_n, w1_ref[h, D:, :], preferred_element_type=jnp.float32)
    b1 = b1_ref[h:h + 1, :].reshape(1, 1, D)
    a1 = _selu(a_e.reshape(K, _TB, D) + a_u[None, :, :] + b1)
    a2 = _selu(jnp.dot(a1.reshape(K * _TB, D), w2_ref[h],
                       preferred_element_type=jnp.float32)
               + b2_ref[h:h + 1, :])
    w3 = w3t_ref[h].reshape(1, 1, D4)
    return jnp.sum(a2.reshape(K, _TB, D4) * w3, axis=-1) + b3_ref[h:h + 1, 0:1]


def _aggregate(en3, attT):
    att = jnp.transpose(attT)
    u = jnp.zeros((_TB, D), jnp.float32)
    for k in range(K):
        u = u + en3[k] * att[:, k:k + 1]
    return u


def _s1_body(e_ref, s_ref, w1_ref, b1_ref, w2_ref, b2_ref, w3t_ref, b3_ref,
             l1t_ref, l1b_ref, en_ref, s1_ref, w1o_ref, w2o_ref):
    en = _normalize_rows(e_ref[...].reshape(K * _TB, D))
    en3 = en.reshape(K, _TB, D)
    en_ref[...] = en3
    for h in range(H):
        l1 = l1t_ref[h].reshape(1, 1, D)
        wT = jax.nn.sigmoid(jnp.sum(en3 * l1, axis=-1)
                            + l1b_ref[h:h + 1, 0:1]) + 1.0
        if h == 0:
            w1o_ref[...] = wT
        else:
            w2o_ref[...] = wT
    s1_ref[...] = _hop_scores(en, en3, s_ref[...], 0, w1_ref, b1_ref, w2_ref,
                              b2_ref, w3t_ref, b3_ref)


def _s1(e_kmaj, sf, att1_W, att1_b, att2_W, att2_b, att3_Wt, att3_b, lin1_Wt,
        lin1_b):
    return pl.pallas_call(
        _s1_body,
        grid=(B // _TB,),
        in_specs=[
            pl.BlockSpec((K, _TB, D), lambda i: (0, i, 0)),
            pl.BlockSpec((_TB, D), lambda i: (i, 0)),
            _full_spec((H, 2 * D, D)),
            _full_spec((H, D)),
            _full_spec((H, D, D4)),
            _full_spec((H, D4)),
            _full_spec((H, 1, D4)),
            _full_spec((H, 1)),
            _full_spec((H, 1, D)),
            _full_spec((H, 1)),
        ],
        out_specs=[
            pl.BlockSpec((K, _TB, D), lambda i: (0, i, 0)),
            pl.BlockSpec((K, _TB), lambda i: (0, i)),
            pl.BlockSpec((K, _TB), lambda i: (0, i)),
            pl.BlockSpec((K, _TB), lambda i: (0, i)),
        ],
        out_shape=[
            jax.ShapeDtypeStruct((K, B, D), jnp.float32),
            jax.ShapeDtypeStruct((K, B), jnp.float32),
            jax.ShapeDtypeStruct((K, B), jnp.float32),
            jax.ShapeDtypeStruct((K, B), jnp.float32),
        ],
    )(e_kmaj, sf, att1_W, att1_b, att2_W, att2_b, att3_Wt, att3_b, lin1_Wt,
      lin1_b)


def _entmax_body(s_ref, w_ref, out_ref):
    x = s_ref[...]
    alpha = w_ref[...]
    am1 = alpha - 1.0
    xs = x * am1
    inv = 1.0 / am1
    mx = jnp.max(xs, axis=0, keepdims=True)
    lo = mx - 1.0
    hi = mx
    for _ in range(N_ITER):
        mid = 0.5 * (lo + hi)
        f = jnp.sum(_safe_pow(jnp.maximum(xs - mid, 0.0), inv), axis=0,
                    keepdims=True) - 1.0
        ge = f >= 0.0
        lo = jnp.where(ge, mid, lo)
        hi = jnp.where(ge, hi, mid)
    tau = 0.5 * (lo + hi)
    p = _safe_pow(jnp.maximum(xs - tau, 0.0), inv)
    out_ref[...] = p / jnp.sum(p, axis=0, keepdims=True)


def _entmax_fb(sT, wT):
    return pl.pallas_call(
        _entmax_body,
        out_shape=jax.ShapeDtypeStruct((K, B), jnp.float32),
    )(sT, wT)


def _a1s2_body(en_ref, att_ref, w1_ref, b1_ref, w2_ref, b2_ref, w3t_ref,
               b3_ref, u1_ref, s2_ref):
    en3 = en_ref[...]
    en = en3.reshape(K * _TB, D)
    u1 = _aggregate(en3, att_ref[...])
    u1_ref[...] = u1
    s2_ref[...] = _hop_scores(en, en3, u1, 1, w1_ref, b1_ref, w2_ref, b2_ref,
                              w3t_ref, b3_ref)


def _a1s2(en, attT1, att1_W, att1_b, att2_W, att2_b, att3_Wt, att3_b):
    return pl.pallas_call(
        _a1s2_body,
        grid=(B // _TB,),
        in_specs=[
            pl.BlockSpec((K, _TB, D), lambda i: (0, i, 0)),
            pl.BlockSpec((K, _TB), lambda i: (0, i)),
            _full_spec((H, 2 * D, D)),
            _full_spec((H, D)),
            _full_spec((H, D, D4)),
            _full_spec((H, D4)),
            _full_spec((H, 1, D4)),
            _full_spec((H, 1)),
        ],
        out_specs=[
            pl.BlockSpec((_TB, D), lambda i: (i, 0)),
            pl.BlockSpec((K, _TB), lambda i: (0, i)),
        ],
        out_shape=[
            jax.ShapeDtypeStruct((B, D), jnp.float32),
            jax.ShapeDtypeStruct((K, B), jnp.float32),
        ],
    )(en, attT1, att1_W, att1_b, att2_W, att2_b, att3_Wt, att3_b)


def _a2_body(en_ref, att_ref, u1_ref, nf_ref):
    u2 = _aggregate(en_ref[...], att_ref[...])
    nf_ref[...] = (u1_ref[...] + u2) * 0.5


def _a2(en, attT2, u1):
    return pl.pallas_call(
        _a2_body,
        grid=(B // _TB,),
        in_specs=[
            pl.BlockSpec((K, _TB, D), lambda i: (0, i, 0)),
            pl.BlockSpec((K, _TB), lambda i: (0, i)),
            pl.BlockSpec((_TB, D), lambda i: (i, 0)),
        ],
        out_specs=pl.BlockSpec((_TB, D), lambda i: (i, 0)),
        out_shape=jax.ShapeDtypeStruct((B, D), jnp.float32),
    )(en, attT2, u1)


def _head_body(nf_ref, sf_ref, inw_ref, inb_ref, outw_ref, outb_ref, gw_ref,
               gb_ref, bng_ref, bnb_ref, bn1g_ref, bn1b_ref, o_ref):
    nf = nf_ref[...]
    sf = sf_ref[...]
    mu = jnp.mean(nf, axis=0, keepdims=True)
    xc = nf - mu
    var = jnp.mean(xc * xc, axis=0, keepdims=True)
    h0 = xc / jnp.sqrt(var + 1e-5) * bng_ref[...] + bnb_ref[...]
    h1 = _selu(jnp.dot(h0, inw_ref[...], preferred_element_type=jnp.float32)
               + inb_ref[...])
    mu1 = jnp.mean(h1, axis=0, keepdims=True)
    xc1 = h1 - mu1
    var1 = jnp.mean(xc1 * xc1, axis=0, keepdims=True)
    h1n = xc1 / jnp.sqrt(var1 + 1e-5) * bn1g_ref[...] + bn1b_ref[...]
    no = jnp.dot(h1n, outw_ref[...], preferred_element_type=jnp.float32) \
        + outb_ref[...]
    z = (jnp.dot(sf, gw_ref[0:D, :], preferred_element_type=jnp.float32)
         + jnp.dot(no, gw_ref[D:2 * D, :], preferred_element_type=jnp.float32)
         + jnp.dot(sf * no, gw_ref[2 * D:, :],
                   preferred_element_type=jnp.float32)
         + gb_ref[...])
    gama = jax.nn.sigmoid(z)
    o_ref[...] = gama * sf + (1.0 - gama) * no


def _head(nf, sf, in_W, in_b, out_W, out_b, gate_W, gate_b, bn_g, bn_b,
          bn1_g, bn1_b):
    return pl.pallas_call(
        _head_body,
        out_shape=jax.ShapeDtypeStruct((B, D), jnp.float32),
    )(nf, sf, in_W, in_b, out_W, out_b, gate_W, gate_b, bn_g, bn_b,
      bn1_g, bn1_b)


def kernel(nodes, neighbors, u2e, att1_W, att1_b, att2_W, att2_b, att3_W,
           att3_b, lin1_W, lin1_b, gate_W, gate_b, bn_g, bn_b, in_W, in_b,
           bn1_g, bn1_b, out_W, out_b):
    idx = jnp.concatenate([neighbors.T.reshape(-1).astype(jnp.int32),
                           nodes.astype(jnp.int32)])
    g = _sc_gather(u2e, idx)
    e_kmaj = g[:B * K].reshape(K, B, D)
    sf = g[B * K:]
    att3_Wt = jnp.transpose(att3_W, (0, 2, 1))
    lin1_Wt = jnp.transpose(lin1_W, (0, 2, 1))
    en, sT1, wT1, wT2 = _s1(e_kmaj, sf, att1_W, att1_b, att2_W, att2_b,
                            att3_Wt, att3_b, lin1_Wt, lin1_b)
    attT1 = _entmax_fb(sT1, wT1)
    u1, sT2 = _a1s2(en, attT1, att1_W, att1_b, att2_W, att2_b, att3_Wt,
                    att3_b)
    attT2 = _entmax_fb(sT2, wT2)
    nf = _a2(en, attT2, u1)
    return _head(nf, sf, in_W, in_b.reshape(1, D), out_W, out_b.reshape(1, D),
                 gate_W, gate_b.reshape(1, D), bn_g.reshape(1, D),
                 bn_b.reshape(1, D), bn1_g.reshape(1, D), bn1_b.reshape(1, D))


# speedup vs baseline: 7.0336x; 1.0213x over previous
import jax
import jax.numpy as jnp
from jax import lax
from jax.experimental import pallas as pl
from jax.experimental.pallas import tpu as pltpu
from jax.experimental.pallas import tpu_sc as plsc

D = 128
B = 4096
K = 32
H = 2
D4 = 32
N_ITER = 30

_SELU_ALPHA = 1.6732632423543772
_SELU_SCALE = 1.0507009873554805

_NC, _NS = 2, 16
_NW = _NC * _NS
_NIDX = B * K + B
_RPW = _NIDX // _NW
_CH = 352
_NCHUNK = _RPW // _CH


def _sc_gather_body(table, idx_hbm, out_hbm, idx_v, buf0, buf1, gs0, gs1,
                    ws0, ws1):
    wid = lax.axis_index("s") * _NC + lax.axis_index("c")
    base = wid * _RPW
    pltpu.sync_copy(idx_hbm.at[pl.ds(base, _RPW)], idx_v)
    bufs = (buf0, buf1)
    gsems = (gs0, gs1)
    wsems = (ws0, ws1)
    gcp = [None, None]
    wcp = [None, None]
    for j in range(_NCHUNK):
        b = j % 2
        if wcp[b] is not None:
            wcp[b].wait()
        gcp[b] = pltpu.async_copy(
            table.at[idx_v.at[pl.ds(j * _CH, _CH)]], bufs[b], gsems[b])
        if j > 0:
            pb = (j - 1) % 2
            gcp[pb].wait()
            wcp[pb] = pltpu.async_copy(
                bufs[pb], out_hbm.at[pl.ds(base + (j - 1) * _CH, _CH)],
                wsems[pb])
    lb = (_NCHUNK - 1) % 2
    gcp[lb].wait()
    wcp[lb] = pltpu.async_copy(
        bufs[lb], out_hbm.at[pl.ds(base + (_NCHUNK - 1) * _CH, _CH)],
        wsems[lb])
    wcp[0].wait()
    wcp[1].wait()


def _sc_gather(u2e, idx):
    f = pl.kernel(
        _sc_gather_body,
        mesh=plsc.VectorSubcoreMesh(core_axis_name="c", subcore_axis_name="s"),
        out_type=jax.ShapeDtypeStruct((_NIDX, D), jnp.float32),
        scratch_types=[
            pltpu.VMEM((_RPW,), jnp.int32),
            pltpu.VMEM((_CH, D), jnp.float32),
            pltpu.VMEM((_CH, D), jnp.float32),
            pltpu.SemaphoreType.DMA,
            pltpu.SemaphoreType.DMA,
            pltpu.SemaphoreType.DMA,
            pltpu.SemaphoreType.DMA,
        ],
    )
    return f(u2e, idx)


_TB = 256


def _normalize_rows(x):
    n = jnp.sqrt(jnp.sum(x * x, axis=-1, keepdims=True))
    return x / jnp.maximum(n, 1e-12)


def _selu(x):
    return _SELU_SCALE * jnp.where(x > 0, x, _SELU_ALPHA * (jnp.exp(x) - 1.0))


def _safe_pow(t, inv):
    pos = t > 0.0
    lg = jnp.log2(jnp.where(pos, t, 1.0))
    return jnp.where(pos, jnp.exp2(inv * lg), 0.0)


def _full_spec(shape):
    return pl.BlockSpec(shape, lambda i: tuple(0 for _ in shape))


def _hop_scores(en, en3, u, h, w1_ref, b1_ref, w2_ref, b2_ref, w3t_ref,
                b3_ref):
    u_n = _normalize_rows(u)
    a_e = jnp.dot(en, w1_ref[h, :D, :], preferred_element_type=jnp.float32)
    a_u = jnp.dot(u_n, w1_ref[h, D:, :], preferred_element_type=jnp.float32)
    b1 = b1_ref[h:h + 1, :].reshape(1, 1, D)
    a1 = _selu(a_e.reshape(K, _TB, D) + a_u[None, :, :] + b1)
    a2 = _selu(jnp.dot(a1.reshape(K * _TB, D), w2_ref[h],
                       preferred_element_type=jnp.float32)
               + b2_ref[h:h + 1, :])
    w3 = w3t_ref[h].reshape(1, 1, D4)
    return jnp.sum(a2.reshape(K, _TB, D4) * w3, axis=-1) + b3_ref[h:h + 1, 0:1]


def _aggregate(en3, attT):
    att = jnp.transpose(attT)
    u = jnp.zeros((_TB, D), jnp.float32)
    for k in range(K):
        u = u + en3[k] * att[:, k:k + 1]
    return u


def _s1_body(e_ref, s_ref, w1_ref, b1_ref, w2_ref, b2_ref, w3t_ref, b3_ref,
             l1t_ref, l1b_ref, en_ref, s1_ref, w1o_ref, w2o_ref):
    en = _normalize_rows(e_ref[...].reshape(K * _TB, D))
    en3 = en.reshape(K, _TB, D)
    en_ref[...] = en3
    for h in range(H):
        l1 = l1t_ref[h].reshape(1, 1, D)
        wT = jax.nn.sigmoid(jnp.sum(en3 * l1, axis=-1)
                            + l1b_ref[h:h + 1, 0:1]) + 1.0
        if h == 0:
            w1o_ref[...] = wT
        else:
            w2o_ref[...] = wT
    s1_ref[...] = _hop_scores(en, en3, s_ref[...], 0, w1_ref, b1_ref, w2_ref,
                              b2_ref, w3t_ref, b3_ref)


def _s1(e_kmaj, sf, att1_W, att1_b, att2_W, att2_b, att3_Wt, att3_b, lin1_Wt,
        lin1_b):
    return pl.pallas_call(
        _s1_body,
        grid=(B // _TB,),
        in_specs=[
            pl.BlockSpec((K, _TB, D), lambda i: (0, i, 0)),
            pl.BlockSpec((_TB, D), lambda i: (i, 0)),
            _full_spec((H, 2 * D, D)),
            _full_spec((H, D)),
            _full_spec((H, D, D4)),
            _full_spec((H, D4)),
            _full_spec((H, 1, D4)),
            _full_spec((H, 1)),
            _full_spec((H, 1, D)),
            _full_spec((H, 1)),
        ],
        out_specs=[
            pl.BlockSpec((K, _TB, D), lambda i: (0, i, 0)),
            pl.BlockSpec((K, _TB), lambda i: (0, i)),
            pl.BlockSpec((K, _TB), lambda i: (0, i)),
            pl.BlockSpec((K, _TB), lambda i: (0, i)),
        ],
        out_shape=[
            jax.ShapeDtypeStruct((K, B, D), jnp.float32),
            jax.ShapeDtypeStruct((K, B), jnp.float32),
            jax.ShapeDtypeStruct((K, B), jnp.float32),
            jax.ShapeDtypeStruct((K, B), jnp.float32),
        ],
    )(e_kmaj, sf, att1_W, att1_b, att2_W, att2_b, att3_Wt, att3_b, lin1_Wt,
      lin1_b)


def _entmax_body(s_ref, w_ref, out_ref):
    x = s_ref[...]
    alpha = w_ref[...]
    am1 = alpha - 1.0
    xs = x * am1
    inv = 1.0 / am1
    mx = jnp.max(xs, axis=0, keepdims=True)
    lo = mx - 1.0
    hi = mx
    for _ in range(N_ITER):
        mid = 0.5 * (lo + hi)
        f = jnp.sum(_safe_pow(jnp.maximum(xs - mid, 0.0), inv), axis=0,
                    keepdims=True) - 1.0
        ge = f >= 0.0
        lo = jnp.where(ge, mid, lo)
        hi = jnp.where(ge, hi, mid)
    tau = 0.5 * (lo + hi)
    p = _safe_pow(jnp.maximum(xs - tau, 0.0), inv)
    out_ref[...] = p / jnp.sum(p, axis=0, keepdims=True)


def _entmax_fb(sT, wT):
    return pl.pallas_call(
        _entmax_body,
        out_shape=jax.ShapeDtypeStruct((K, B), jnp.float32),
    )(sT, wT)


def _a1s2_body(en_ref, att_ref, w1_ref, b1_ref, w2_ref, b2_ref, w3t_ref,
               b3_ref, u1_ref, s2_ref):
    en3 = en_ref[...]
    en = en3.reshape(K * _TB, D)
    u1 = _aggregate(en3, att_ref[...])
    u1_ref[...] = u1
    s2_ref[...] = _hop_scores(en, en3, u1, 1, w1_ref, b1_ref, w2_ref, b2_ref,
                              w3t_ref, b3_ref)


def _a1s2(en, attT1, att1_W, att1_b, att2_W, att2_b, att3_Wt, att3_b):
    return pl.pallas_call(
        _a1s2_body,
        grid=(B // _TB,),
        in_specs=[
            pl.BlockSpec((K, _TB, D), lambda i: (0, i, 0)),
            pl.BlockSpec((K, _TB), lambda i: (0, i)),
            _full_spec((H, 2 * D, D)),
            _full_spec((H, D)),
            _full_spec((H, D, D4)),
            _full_spec((H, D4)),
            _full_spec((H, 1, D4)),
            _full_spec((H, 1)),
        ],
        out_specs=[
            pl.BlockSpec((_TB, D), lambda i: (i, 0)),
            pl.BlockSpec((K, _TB), lambda i: (0, i)),
        ],
        out_shape=[
            jax.ShapeDtypeStruct((B, D), jnp.float32),
            jax.ShapeDtypeStruct((K, B), jnp.float32),
        ],
    )(en, attT1, att1_W, att1_b, att2_W, att2_b, att3_Wt, att3_b)


def _a2_body(en_ref, att_ref, u1_ref, nf_ref):
    u2 = _aggregate(en_ref[...], att_ref[...])
    nf_ref[...] = (u1_ref[...] + u2) * 0.5


def _a2(en, attT2, u1):
    return pl.pallas_call(
        _a2_body,
        grid=(B // _TB,),
        in_specs=[
            pl.BlockSpec((K, _TB, D), lambda i: (0, i, 0)),
            pl.BlockSpec((K, _TB), lambda i: (0, i)),
            pl.BlockSpec((_TB, D), lambda i: (i, 0)),
        ],
        out_specs=pl.BlockSpec((_TB, D), lambda i: (i, 0)),
        out_shape=jax.ShapeDtypeStruct((B, D), jnp.float32),
    )(en, attT2, u1)


def _head_body(nf_ref, sf_ref, inw_ref, inb_ref, outw_ref, outb_ref, gw_ref,
               gb_ref, bng_ref, bnb_ref, bn1g_ref, bn1b_ref, o_ref):
    nf = nf_ref[...]
    sf = sf_ref[...]
    mu = jnp.mean(nf, axis=0, keepdims=True)
    xc = nf - mu
    var = jnp.mean(xc * xc, axis=0, keepdims=True)
    h0 = xc / jnp.sqrt(var + 1e-5) * bng_ref[...] + bnb_ref[...]
    h1 = _selu(jnp.dot(h0, inw_ref[...], preferred_element_type=jnp.float32)
               + inb_ref[...])
    mu1 = jnp.mean(h1, axis=0, keepdims=True)
    xc1 = h1 - mu1
    var1 = jnp.mean(xc1 * xc1, axis=0, keepdims=True)
    h1n = xc1 / jnp.sqrt(var1 + 1e-5) * bn1g_ref[...] + bn1b_ref[...]
    no = jnp.dot(h1n, outw_ref[...], preferred_element_type=jnp.float32) \
        + outb_ref[...]
    z = (jnp.dot(sf, gw_ref[0:D, :], preferred_element_type=jnp.float32)
         + jnp.dot(no, gw_ref[D:2 * D, :], preferred_element_type=jnp.float32)
         + jnp.dot(sf * no, gw_ref[2 * D:, :],
                   preferred_element_type=jnp.float32)
         + gb_ref[...])
    gama = jax.nn.sigmoid(z)
    o_ref[...] = gama * sf + (1.0 - gama) * no


def _head(nf, sf, in_W, in_b, out_W, out_b, gate_W, gate_b, bn_g, bn_b,
          bn1_g, bn1_b):
    return pl.pallas_call(
        _head_body,
        out_shape=jax.ShapeDtypeStruct((B, D), jnp.float32),
    )(nf, sf, in_W, in_b, out_W, out_b, gate_W, gate_b, bn_g, bn_b,
      bn1_g, bn1_b)


def kernel(nodes, neighbors, u2e, att1_W, att1_b, att2_W, att2_b, att3_W,
           att3_b, lin1_W, lin1_b, gate_W, gate_b, bn_g, bn_b, in_W, in_b,
           bn1_g, bn1_b, out_W, out_b):
    idx = jnp.concatenate([neighbors.T.reshape(-1).astype(jnp.int32),
                           nodes.astype(jnp.int32)])
    g = _sc_gather(u2e, idx)
    e_kmaj = g[:B * K].reshape(K, B, D)
    sf = g[B * K:]
    att3_Wt = jnp.transpose(att3_W, (0, 2, 1))
    lin1_Wt = jnp.transpose(lin1_W, (0, 2, 1))
    en, sT1, wT1, wT2 = _s1(e_kmaj, sf, att1_W, att1_b, att2_W, att2_b,
                            att3_Wt, att3_b, lin1_Wt, lin1_b)
    attT1 = _entmax_fb(sT1, wT1)
    u1, sT2 = _a1s2(en, attT1, att1_W, att1_b, att2_W, att2_b, att3_Wt,
                    att3_b)
    attT2 = _entmax_fb(sT2, wT2)
    nf = _a2(en, attT2, u1)
    return _head(nf, sf, in_W, in_b.reshape(1, D), out_W, out_b.reshape(1, D),
                 gate_W, gate_b.reshape(1, D), bn_g.reshape(1, D),
                 bn_b.reshape(1, D), bn1_g.reshape(1, D), bn1_b.reshape(1, D))


# speedup vs baseline: 7.0650x; 1.0045x over previous
import jax
import jax.numpy as jnp
from jax import lax
from jax.experimental import pallas as pl
from jax.experimental.pallas import tpu as pltpu
from jax.experimental.pallas import tpu_sc as plsc

D = 128
B = 4096
K = 32
H = 2
D4 = 32
N_ITER = 30

_SELU_ALPHA = 1.6732632423543772
_SELU_SCALE = 1.0507009873554805

_NC, _NS = 2, 16
_NW = _NC * _NS
_NIDX = B * K + B
_RPW = _NIDX // _NW
_CH = 352
_NCHUNK = _RPW // _CH


def _sc_gather_body(table, idx_hbm, out_hbm, idx_v, buf0, buf1, gs0, gs1,
                    ws0, ws1):
    wid = lax.axis_index("s") * _NC + lax.axis_index("c")
    base = wid * _RPW
    pltpu.sync_copy(idx_hbm.at[pl.ds(base, _RPW)], idx_v)
    bufs = (buf0, buf1)
    gsems = (gs0, gs1)
    wsems = (ws0, ws1)
    gcp = [None, None]
    wcp = [None, None]
    for j in range(_NCHUNK):
        b = j % 2
        if wcp[b] is not None:
            wcp[b].wait()
        gcp[b] = pltpu.async_copy(
            table.at[idx_v.at[pl.ds(j * _CH, _CH)]], bufs[b], gsems[b])
        if j > 0:
            pb = (j - 1) % 2
            gcp[pb].wait()
            wcp[pb] = pltpu.async_copy(
                bufs[pb], out_hbm.at[pl.ds(base + (j - 1) * _CH, _CH)],
                wsems[pb])
    lb = (_NCHUNK - 1) % 2
    gcp[lb].wait()
    wcp[lb] = pltpu.async_copy(
        bufs[lb], out_hbm.at[pl.ds(base + (_NCHUNK - 1) * _CH, _CH)],
        wsems[lb])
    wcp[0].wait()
    wcp[1].wait()


def _sc_gather(u2e, idx):
    f = pl.kernel(
        _sc_gather_body,
        mesh=plsc.VectorSubcoreMesh(core_axis_name="c", subcore_axis_name="s"),
        out_type=jax.ShapeDtypeStruct((_NIDX, D), jnp.float32),
        scratch_types=[
            pltpu.VMEM((_RPW,), jnp.int32),
            pltpu.VMEM((_CH, D), jnp.float32),
            pltpu.VMEM((_CH, D), jnp.float32),
            pltpu.SemaphoreType.DMA,
            pltpu.SemaphoreType.DMA,
            pltpu.SemaphoreType.DMA,
            pltpu.SemaphoreType.DMA,
        ],
    )
    return f(u2e, idx)


_TB = 256


def _normalize_rows(x):
    n = jnp.sqrt(jnp.sum(x * x, axis=-1, keepdims=True))
    return x / jnp.maximum(n, 1e-12)


def _selu(x):
    return _SELU_SCALE * jnp.where(x > 0, x, _SELU_ALPHA * (jnp.exp(x) - 1.0))


def _safe_pow(t, inv):
    pos = t > 0.0
    lg = jnp.log2(jnp.where(pos, t, 1.0))
    return jnp.where(pos, jnp.exp2(inv * lg), 0.0)


def _full_spec(shape):
    return pl.BlockSpec(shape, lambda i: tuple(0 for _ in shape))


def _hop_scores(en, en3, u, h, w1_ref, b1_ref, w2_ref, b2_ref, w3t_ref,
                b3_ref):
    u_n = _normalize_rows(u)
    a_e = jnp.dot(en, w1_ref[h, :D, :], preferred_element_type=jnp.float32)
    a_u = jnp.dot(u_n, w1_ref[h, D:, :], preferred_element_type=jnp.float32)
    b1 = b1_ref[h:h + 1, :].reshape(1, 1, D)
    a1 = _selu(a_e.reshape(K, _TB, D) + a_u[None, :, :] + b1)
    a2 = _selu(jnp.dot(a1.reshape(K * _TB, D), w2_ref[h],
                       preferred_element_type=jnp.float32)
               + b2_ref[h:h + 1, :])
    w3 = w3t_ref[h].reshape(1, 1, D4)
    return jnp.sum(a2.reshape(K, _TB, D4) * w3, axis=-1) + b3_ref[h:h + 1, 0:1]


def _aggregate(en3, attT):
    att = jnp.transpose(attT)
    u = jnp.zeros((_TB, D), jnp.float32)
    for k in range(K):
        u = u + en3[k] * att[:, k:k + 1]
    return u


def _s1_body(e_ref, s_ref, w1_ref, b1_ref, w2_ref, b2_ref, w3t_ref, b3_ref,
             l1t_ref, l1b_ref, en_ref, s1_ref, w1o_ref, w2o_ref):
    en = _normalize_rows(e_ref[...].reshape(K * _TB, D))
    en3 = en.reshape(K, _TB, D)
    en_ref[...] = en3
    for h in range(H):
        l1 = l1t_ref[h].reshape(1, 1, D)
        wT = jax.nn.sigmoid(jnp.sum(en3 * l1, axis=-1)
                            + l1b_ref[h:h + 1, 0:1]) + 1.0
        if h == 0:
            w1o_ref[...] = wT
        else:
            w2o_ref[...] = wT
    s1_ref[...] = _hop_scores(en, en3, s_ref[...], 0, w1_ref, b1_ref, w2_ref,
                              b2_ref, w3t_ref, b3_ref)


def _s1(e_kmaj, sf, att1_W, att1_b, att2_W, att2_b, att3_Wt, att3_b, lin1_Wt,
        lin1_b):
    return pl.pallas_call(
        _s1_body,
        grid=(B // _TB,),
        in_specs=[
            pl.BlockSpec((K, _TB, D), lambda i: (0, i, 0)),
            pl.BlockSpec((_TB, D), lambda i: (i, 0)),
            _full_spec((H, 2 * D, D)),
            _full_spec((H, D)),
            _full_spec((H, D, D4)),
            _full_spec((H, D4)),
            _full_spec((H, 1, D4)),
            _full_spec((H, 1)),
            _full_spec((H, 1, D)),
            _full_spec((H, 1)),
        ],
        out_specs=[
            pl.BlockSpec((K, _TB, D), lambda i: (0, i, 0)),
            pl.BlockSpec((K, _TB), lambda i: (0, i)),
            pl.BlockSpec((K, _TB), lambda i: (0, i)),
            pl.BlockSpec((K, _TB), lambda i: (0, i)),
        ],
        out_shape=[
            jax.ShapeDtypeStruct((K, B, D), jnp.float32),
            jax.ShapeDtypeStruct((K, B), jnp.float32),
            jax.ShapeDtypeStruct((K, B), jnp.float32),
            jax.ShapeDtypeStruct((K, B), jnp.float32),
        ],
    )(e_kmaj, sf, att1_W, att1_b, att2_W, att2_b, att3_Wt, att3_b, lin1_Wt,
      lin1_b)


def _entmax_body(s_ref, w_ref, out_ref):
    x = s_ref[...]
    alpha = w_ref[...]
    am1 = alpha - 1.0
    xs = x * am1
    inv = 1.0 / am1
    mx = jnp.max(xs, axis=0, keepdims=True)
    lo = mx - 1.0
    hi = mx
    for _ in range(N_ITER):
        mid = 0.5 * (lo + hi)
        f = jnp.sum(_safe_pow(jnp.maximum(xs - mid, 0.0), inv), axis=0,
                    keepdims=True) - 1.0
        ge = f >= 0.0
        lo = jnp.where(ge, mid, lo)
        hi = jnp.where(ge, hi, mid)
    tau = 0.5 * (lo + hi)
    p = _safe_pow(jnp.maximum(xs - tau, 0.0), inv)
    out_ref[...] = p / jnp.sum(p, axis=0, keepdims=True)


def _entmax_fb(sT, wT):
    return pl.pallas_call(
        _entmax_body,
        out_shape=jax.ShapeDtypeStruct((K, B), jnp.float32),
    )(sT, wT)


def _a1s2_body(en_ref, att_ref, w1_ref, b1_ref, w2_ref, b2_ref, w3t_ref,
               b3_ref, u1_ref, s2_ref):
    en3 = en_ref[...]
    en = en3.reshape(K * _TB, D)
    u1 = _aggregate(en3, att_ref[...])
    u1_ref[...] = u1
    s2_ref[...] = _hop_scores(en, en3, u1, 1, w1_ref, b1_ref, w2_ref, b2_ref,
                              w3t_ref, b3_ref)


def _a1s2(en, attT1, att1_W, att1_b, att2_W, att2_b, att3_Wt, att3_b):
    return pl.pallas_call(
        _a1s2_body,
        grid=(B // _TB,),
        in_specs=[
            pl.BlockSpec((K, _TB, D), lambda i: (0, i, 0)),
            pl.BlockSpec((K, _TB), lambda i: (0, i)),
            _full_spec((H, 2 * D, D)),
            _full_spec((H, D)),
            _full_spec((H, D, D4)),
            _full_spec((H, D4)),
            _full_spec((H, 1, D4)),
            _full_spec((H, 1)),
        ],
        out_specs=[
            pl.BlockSpec((_TB, D), lambda i: (i, 0)),
            pl.BlockSpec((K, _TB), lambda i: (0, i)),
        ],
        out_shape=[
            jax.ShapeDtypeStruct((B, D), jnp.float32),
            jax.ShapeDtypeStruct((K, B), jnp.float32),
        ],
    )(en, attT1, att1_W, att1_b, att2_W, att2_b, att3_Wt, att3_b)


def _a2_body(en_ref, att_ref, u1_ref, nf_ref):
    u2 = _aggregate(en_ref[...], att_ref[...])
    nf_ref[...] = (u1_ref[...] + u2) * 0.5


def _a2(en, attT2, u1):
    return pl.pallas_call(
        _a2_body,
        grid=(B // _TB,),
        in_specs=[
            pl.BlockSpec((K, _TB, D), lambda i: (0, i, 0)),
            pl.BlockSpec((K, _TB), lambda i: (0, i)),
            pl.BlockSpec((_TB, D), lambda i: (i, 0)),
        ],
        out_specs=pl.BlockSpec((_TB, D), lambda i: (i, 0)),
        out_shape=jax.ShapeDtypeStruct((B, D), jnp.float32),
    )(en, attT2, u1)


def _bmean(x):
    s = jnp.sum(x.reshape(32, B // 32, D), axis=0)
    return jnp.sum(s, axis=0, keepdims=True) * (1.0 / B)


def _head_body(nf_ref, sf_ref, inw_ref, inb_ref, outw_ref, outb_ref, gw_ref,
               gb_ref, bng_ref, bnb_ref, bn1g_ref, bn1b_ref, o_ref):
    nf = nf_ref[...]
    sf = sf_ref[...]
    mu = _bmean(nf)
    xc = nf - mu
    var = _bmean(xc * xc)
    h0 = xc / jnp.sqrt(var + 1e-5) * bng_ref[...] + bnb_ref[...]
    h1 = _selu(jnp.dot(h0, inw_ref[...], preferred_element_type=jnp.float32)
               + inb_ref[...])
    mu1 = _bmean(h1)
    xc1 = h1 - mu1
    var1 = _bmean(xc1 * xc1)
    h1n = xc1 / jnp.sqrt(var1 + 1e-5) * bn1g_ref[...] + bn1b_ref[...]
    no = jnp.dot(h1n, outw_ref[...], preferred_element_type=jnp.float32) \
        + outb_ref[...]
    z = (jnp.dot(sf, gw_ref[0:D, :], preferred_element_type=jnp.float32)
         + jnp.dot(no, gw_ref[D:2 * D, :], preferred_element_type=jnp.float32)
         + jnp.dot(sf * no, gw_ref[2 * D:, :],
                   preferred_element_type=jnp.float32)
         + gb_ref[...])
    gama = jax.nn.sigmoid(z)
    o_ref[...] = gama * sf + (1.0 - gama) * no


def _head(nf, sf, in_W, in_b, out_W, out_b, gate_W, gate_b, bn_g, bn_b,
          bn1_g, bn1_b):
    return pl.pallas_call(
        _head_body,
        out_shape=jax.ShapeDtypeStruct((B, D), jnp.float32),
    )(nf, sf, in_W, in_b, out_W, out_b, gate_W, gate_b, bn_g, bn_b,
      bn1_g, bn1_b)


def kernel(nodes, neighbors, u2e, att1_W, att1_b, att2_W, att2_b, att3_W,
           att3_b, lin1_W, lin1_b, gate_W, gate_b, bn_g, bn_b, in_W, in_b,
           bn1_g, bn1_b, out_W, out_b):
    idx = jnp.concatenate([neighbors.T.reshape(-1).astype(jnp.int32),
                           nodes.astype(jnp.int32)])
    g = _sc_gather(u2e, idx)
    e_kmaj = g[:B * K].reshape(K, B, D)
    sf = g[B * K:]
    att3_Wt = jnp.transpose(att3_W, (0, 2, 1))
    lin1_Wt = jnp.transpose(lin1_W, (0, 2, 1))
    en, sT1, wT1, wT2 = _s1(e_kmaj, sf, att1_W, att1_b, att2_W, att2_b,
                            att3_Wt, att3_b, lin1_Wt, lin1_b)
    attT1 = _entmax_fb(sT1, wT1)
    u1, sT2 = _a1s2(en, attT1, att1_W, att1_b, att2_W, att2_b, att3_Wt,
                    att3_b)
    attT2 = _entmax_fb(sT2, wT2)
    nf = _a2(en, attT2, u1)
    return _head(nf, sf, in_W, in_b.reshape(1, D), out_W, out_b.reshape(1, D),
                 gate_W, gate_b.reshape(1, D), bn_g.reshape(1, D),
                 bn_b.reshape(1, D), bn1_g.reshape(1, D), bn1_b.reshape(1, D))


# speedup vs baseline: 7.0892x; 1.0034x over previous
import jax
import jax.numpy as jnp
from jax import lax
from jax.experimental import pallas as pl
from jax.experimental.pallas import tpu as pltpu
from jax.experimental.pallas import tpu_sc as plsc

D = 128
B = 4096
K = 32
H = 2
D4 = 32
N_ITER = 30

_SELU_ALPHA = 1.6732632423543772
_SELU_SCALE = 1.0507009873554805

_NC, _NS = 2, 16
_NW = _NC * _NS
_NIDX = B * K + B
_RPW = _NIDX // _NW
_CH = 352
_NCHUNK = _RPW // _CH


def _sc_gather_body(table, idx_hbm, out_hbm, idx_v, buf0, buf1, gs0, gs1,
                    ws0, ws1):
    wid = lax.axis_index("s") * _NC + lax.axis_index("c")
    base = wid * _RPW
    pltpu.sync_copy(idx_hbm.at[pl.ds(base, _RPW)], idx_v)
    bufs = (buf0, buf1)
    gsems = (gs0, gs1)
    wsems = (ws0, ws1)
    gcp = [None, None]
    wcp = [None, None]
    for j in range(_NCHUNK):
        b = j % 2
        if wcp[b] is not None:
            wcp[b].wait()
        gcp[b] = pltpu.async_copy(
            table.at[idx_v.at[pl.ds(j * _CH, _CH)]], bufs[b], gsems[b])
        if j > 0:
            pb = (j - 1) % 2
            gcp[pb].wait()
            wcp[pb] = pltpu.async_copy(
                bufs[pb], out_hbm.at[pl.ds(base + (j - 1) * _CH, _CH)],
                wsems[pb])
    lb = (_NCHUNK - 1) % 2
    gcp[lb].wait()
    wcp[lb] = pltpu.async_copy(
        bufs[lb], out_hbm.at[pl.ds(base + (_NCHUNK - 1) * _CH, _CH)],
        wsems[lb])
    wcp[0].wait()
    wcp[1].wait()


def _sc_gather(u2e, idx):
    f = pl.kernel(
        _sc_gather_body,
        mesh=plsc.VectorSubcoreMesh(core_axis_name="c", subcore_axis_name="s"),
        out_type=jax.ShapeDtypeStruct((_NIDX, D), jnp.float32),
        scratch_types=[
            pltpu.VMEM((_RPW,), jnp.int32),
            pltpu.VMEM((_CH, D), jnp.float32),
            pltpu.VMEM((_CH, D), jnp.float32),
            pltpu.SemaphoreType.DMA,
            pltpu.SemaphoreType.DMA,
            pltpu.SemaphoreType.DMA,
            pltpu.SemaphoreType.DMA,
        ],
    )
    return f(u2e, idx)


_TB_S1 = 256
_TB_AG = 512


def _normalize_rows(x):
    n = jnp.sqrt(jnp.sum(x * x, axis=-1, keepdims=True))
    return x / jnp.maximum(n, 1e-12)


def _selu(x):
    return _SELU_SCALE * jnp.where(x > 0, x, _SELU_ALPHA * (jnp.exp(x) - 1.0))


def _safe_pow(t, inv):
    pos = t > 0.0
    lg = jnp.log2(jnp.where(pos, t, 1.0))
    return jnp.where(pos, jnp.exp2(inv * lg), 0.0)


def _full_spec(shape):
    return pl.BlockSpec(shape, lambda i: tuple(0 for _ in shape))


def _hop_scores(en, en3, u, h, w1_ref, b1_ref, w2_ref, b2_ref, w3t_ref,
                b3_ref):
    tb = en.shape[0] // K
    u_n = _normalize_rows(u)
    a_e = jnp.dot(en, w1_ref[h, :D, :], preferred_element_type=jnp.float32)
    a_u = jnp.dot(u_n, w1_ref[h, D:, :], preferred_element_type=jnp.float32)
    b1 = b1_ref[h:h + 1, :].reshape(1, 1, D)
    a1 = _selu(a_e.reshape(K, tb, D) + a_u[None, :, :] + b1)
    a2 = _selu(jnp.dot(a1.reshape(K * tb, D), w2_ref[h],
                       preferred_element_type=jnp.float32)
               + b2_ref[h:h + 1, :])
    w3 = w3t_ref[h].reshape(1, 1, D4)
    return jnp.sum(a2.reshape(K, tb, D4) * w3, axis=-1) + b3_ref[h:h + 1, 0:1]


def _aggregate(en3, attT):
    att = jnp.transpose(attT)
    u = jnp.zeros((en3.shape[1], D), jnp.float32)
    for k in range(K):
        u = u + en3[k] * att[:, k:k + 1]
    return u


def _s1_body(e_ref, s_ref, w1_ref, b1_ref, w2_ref, b2_ref, w3t_ref, b3_ref,
             l1t_ref, l1b_ref, en_ref, s1_ref, w1o_ref, w2o_ref):
    tb = e_ref.shape[1]
    en = _normalize_rows(e_ref[...].reshape(K * tb, D))
    en3 = en.reshape(K, tb, D)
    en_ref[...] = en3
    for h in range(H):
        l1 = l1t_ref[h].reshape(1, 1, D)
        wT = jax.nn.sigmoid(jnp.sum(en3 * l1, axis=-1)
                            + l1b_ref[h:h + 1, 0:1]) + 1.0
        if h == 0:
            w1o_ref[...] = wT
        else:
            w2o_ref[...] = wT
    s1_ref[...] = _hop_scores(en, en3, s_ref[...], 0, w1_ref, b1_ref, w2_ref,
                              b2_ref, w3t_ref, b3_ref)


def _s1(e_kmaj, sf, att1_W, att1_b, att2_W, att2_b, att3_Wt, att3_b, lin1_Wt,
        lin1_b):
    return pl.pallas_call(
        _s1_body,
        grid=(B // _TB_S1,),
        in_specs=[
            pl.BlockSpec((K, _TB_S1, D), lambda i: (0, i, 0)),
            pl.BlockSpec((_TB_S1, D), lambda i: (i, 0)),
            _full_spec((H, 2 * D, D)),
            _full_spec((H, D)),
            _full_spec((H, D, D4)),
            _full_spec((H, D4)),
            _full_spec((H, 1, D4)),
            _full_spec((H, 1)),
            _full_spec((H, 1, D)),
            _full_spec((H, 1)),
        ],
        out_specs=[
            pl.BlockSpec((K, _TB_S1, D), lambda i: (0, i, 0)),
            pl.BlockSpec((K, _TB_S1), lambda i: (0, i)),
            pl.BlockSpec((K, _TB_S1), lambda i: (0, i)),
            pl.BlockSpec((K, _TB_S1), lambda i: (0, i)),
        ],
        out_shape=[
            jax.ShapeDtypeStruct((K, B, D), jnp.float32),
            jax.ShapeDtypeStruct((K, B), jnp.float32),
            jax.ShapeDtypeStruct((K, B), jnp.float32),
            jax.ShapeDtypeStruct((K, B), jnp.float32),
        ],
    )(e_kmaj, sf, att1_W, att1_b, att2_W, att2_b, att3_Wt, att3_b, lin1_Wt,
      lin1_b)


def _entmax_body(s_ref, w_ref, out_ref):
    x = s_ref[...]
    alpha = w_ref[...]
    am1 = alpha - 1.0
    xs = x * am1
    inv = 1.0 / am1
    mx = jnp.max(xs, axis=0, keepdims=True)
    lo = mx - 1.0
    hi = mx
    for _ in range(N_ITER):
        mid = 0.5 * (lo + hi)
        f = jnp.sum(_safe_pow(jnp.maximum(xs - mid, 0.0), inv), axis=0,
                    keepdims=True) - 1.0
        ge = f >= 0.0
        lo = jnp.where(ge, mid, lo)
        hi = jnp.where(ge, hi, mid)
    tau = 0.5 * (lo + hi)
    p = _safe_pow(jnp.maximum(xs - tau, 0.0), inv)
    out_ref[...] = p / jnp.sum(p, axis=0, keepdims=True)


def _entmax_fb(sT, wT):
    return pl.pallas_call(
        _entmax_body,
        out_shape=jax.ShapeDtypeStruct((K, B), jnp.float32),
    )(sT, wT)


def _a1s2_body(en_ref, att_ref, w1_ref, b1_ref, w2_ref, b2_ref, w3t_ref,
               b3_ref, u1_ref, s2_ref):
    en3 = en_ref[...]
    en = en3.reshape(K * en3.shape[1], D)
    u1 = _aggregate(en3, att_ref[...])
    u1_ref[...] = u1
    s2_ref[...] = _hop_scores(en, en3, u1, 1, w1_ref, b1_ref, w2_ref, b2_ref,
                              w3t_ref, b3_ref)


def _a1s2(en, attT1, att1_W, att1_b, att2_W, att2_b, att3_Wt, att3_b):
    return pl.pallas_call(
        _a1s2_body,
        grid=(B // _TB_AG,),
        in_specs=[
            pl.BlockSpec((K, _TB_AG, D), lambda i: (0, i, 0)),
            pl.BlockSpec((K, _TB_AG), lambda i: (0, i)),
            _full_spec((H, 2 * D, D)),
            _full_spec((H, D)),
            _full_spec((H, D, D4)),
            _full_spec((H, D4)),
            _full_spec((H, 1, D4)),
            _full_spec((H, 1)),
        ],
        out_specs=[
            pl.BlockSpec((_TB_AG, D), lambda i: (i, 0)),
            pl.BlockSpec((K, _TB_AG), lambda i: (0, i)),
        ],
        out_shape=[
            jax.ShapeDtypeStruct((B, D), jnp.float32),
            jax.ShapeDtypeStruct((K, B), jnp.float32),
        ],
    )(en, attT1, att1_W, att1_b, att2_W, att2_b, att3_Wt, att3_b)


def _a2_body(en_ref, att_ref, u1_ref, nf_ref):
    u2 = _aggregate(en_ref[...], att_ref[...])
    nf_ref[...] = (u1_ref[...] + u2) * 0.5


def _a2(en, attT2, u1):
    return pl.pallas_call(
        _a2_body,
        grid=(B // _TB_AG,),
        in_specs=[
            pl.BlockSpec((K, _TB_AG, D), lambda i: (0, i, 0)),
            pl.BlockSpec((K, _TB_AG), lambda i: (0, i)),
            pl.BlockSpec((_TB_AG, D), lambda i: (i, 0)),
        ],
        out_specs=pl.BlockSpec((_TB_AG, D), lambda i: (i, 0)),
        out_shape=jax.ShapeDtypeStruct((B, D), jnp.float32),
    )(en, attT2, u1)


def _bmean(x):
    s = jnp.sum(x.reshape(32, B // 32, D), axis=0)
    return jnp.sum(s, axis=0, keepdims=True) * (1.0 / B)


def _head_body(nf_ref, sf_ref, inw_ref, inb_ref, outw_ref, outb_ref, gw_ref,
               gb_ref, bng_ref, bnb_ref, bn1g_ref, bn1b_ref, o_ref):
    nf = nf_ref[...]
    sf = sf_ref[...]
    mu = _bmean(nf)
    xc = nf - mu
    var = _bmean(xc * xc)
    h0 = xc / jnp.sqrt(var + 1e-5) * bng_ref[...] + bnb_ref[...]
    h1 = _selu(jnp.dot(h0, inw_ref[...], preferred_element_type=jnp.float32)
               + inb_ref[...])
    mu1 = _bmean(h1)
    xc1 = h1 - mu1
    var1 = _bmean(xc1 * xc1)
    h1n = xc1 / jnp.sqrt(var1 + 1e-5) * bn1g_ref[...] + bn1b_ref[...]
    no = jnp.dot(h1n, outw_ref[...], preferred_element_type=jnp.float32) \
        + outb_ref[...]
    z = (jnp.dot(sf, gw_ref[0:D, :], preferred_element_type=jnp.float32)
         + jnp.dot(no, gw_ref[D:2 * D, :], preferred_element_type=jnp.float32)
         + jnp.dot(sf * no, gw_ref[2 * D:, :],
                   preferred_element_type=jnp.float32)
         + gb_ref[...])
    gama = jax.nn.sigmoid(z)
    o_ref[...] = gama * sf + (1.0 - gama) * no


def _head(nf, sf, in_W, in_b, out_W, out_b, gate_W, gate_b, bn_g, bn_b,
          bn1_g, bn1_b):
    return pl.pallas_call(
        _head_body,
        out_shape=jax.ShapeDtypeStruct((B, D), jnp.float32),
    )(nf, sf, in_W, in_b, out_W, out_b, gate_W, gate_b, bn_g, bn_b,
      bn1_g, bn1_b)


def kernel(nodes, neighbors, u2e, att1_W, att1_b, att2_W, att2_b, att3_W,
           att3_b, lin1_W, lin1_b, gate_W, gate_b, bn_g, bn_b, in_W, in_b,
           bn1_g, bn1_b, out_W, out_b):
    idx = jnp.concatenate([neighbors.T.reshape(-1).astype(jnp.int32),
                           nodes.astype(jnp.int32)])
    g = _sc_gather(u2e, idx)
    e_kmaj = g[:B * K].reshape(K, B, D)
    sf = g[B * K:]
    att3_Wt = jnp.transpose(att3_W, (0, 2, 1))
    lin1_Wt = jnp.transpose(lin1_W, (0, 2, 1))
    en, sT1, wT1, wT2 = _s1(e_kmaj, sf, att1_W, att1_b, att2_W, att2_b,
                            att3_Wt, att3_b, lin1_Wt, lin1_b)
    attT1 = _entmax_fb(sT1, wT1)
    u1, sT2 = _a1s2(en, attT1, att1_W, att1_b, att2_W, att2_b, att3_Wt,
                    att3_b)
    attT2 = _entmax_fb(sT2, wT2)
    nf = _a2(en, attT2, u1)
    return _head(nf, sf, in_W, in_b.reshape(1, D), out_W, out_b.reshape(1, D),
                 gate_W, gate_b.reshape(1, D), bn_g.reshape(1, D),
                 bn_b.reshape(1, D), bn1_g.reshape(1, D), bn1_b.reshape(1, D))


# speedup vs baseline: 7.1975x; 1.0153x over previous
import jax
import jax.numpy as jnp
from jax import lax
from jax.experimental import pallas as pl
from jax.experimental.pallas import tpu as pltpu
from jax.experimental.pallas import tpu_sc as plsc

D = 128
B = 4096
K = 32
H = 2
D4 = 32
N_ITER = 30

_SELU_ALPHA = 1.6732632423543772
_SELU_SCALE = 1.0507009873554805

_NC, _NS = 2, 16
_NW = _NC * _NS
_NIDX = B * K + B
_RPW = _NIDX // _NW
_CH = 352
_NCHUNK = _RPW // _CH


def _sc_gather_body(table, idx_hbm, out_hbm, idx_v, buf0, buf1, gs0, gs1,
                    ws0, ws1):
    wid = lax.axis_index("s") * _NC + lax.axis_index("c")
    base = wid * _RPW
    pltpu.sync_copy(idx_hbm.at[pl.ds(base, _RPW)], idx_v)
    bufs = (buf0, buf1)
    gsems = (gs0, gs1)
    wsems = (ws0, ws1)
    gcp = [None, None]
    wcp = [None, None]
    for j in range(_NCHUNK):
        b = j % 2
        if wcp[b] is not None:
            wcp[b].wait()
        gcp[b] = pltpu.async_copy(
            table.at[idx_v.at[pl.ds(j * _CH, _CH)]], bufs[b], gsems[b])
        if j > 0:
            pb = (j - 1) % 2
            gcp[pb].wait()
            wcp[pb] = pltpu.async_copy(
                bufs[pb], out_hbm.at[pl.ds(base + (j - 1) * _CH, _CH)],
                wsems[pb])
    lb = (_NCHUNK - 1) % 2
    gcp[lb].wait()
    wcp[lb] = pltpu.async_copy(
        bufs[lb], out_hbm.at[pl.ds(base + (_NCHUNK - 1) * _CH, _CH)],
        wsems[lb])
    wcp[0].wait()
    wcp[1].wait()


def _sc_gather(u2e, idx):
    f = pl.kernel(
        _sc_gather_body,
        mesh=plsc.VectorSubcoreMesh(core_axis_name="c", subcore_axis_name="s"),
        out_type=jax.ShapeDtypeStruct((_NIDX, D), jnp.float32),
        scratch_types=[
            pltpu.VMEM((_RPW,), jnp.int32),
            pltpu.VMEM((_CH, D), jnp.float32),
            pltpu.VMEM((_CH, D), jnp.float32),
            pltpu.SemaphoreType.DMA,
            pltpu.SemaphoreType.DMA,
            pltpu.SemaphoreType.DMA,
            pltpu.SemaphoreType.DMA,
        ],
    )
    return f(u2e, idx)


_TB_S1 = 256
_TB_AG = 512


def _normalize_rows(x):
    n = jnp.sqrt(jnp.sum(x * x, axis=-1, keepdims=True))
    return x / jnp.maximum(n, 1e-12)


def _selu(x):
    return _SELU_SCALE * jnp.where(x > 0, x, _SELU_ALPHA * (jnp.exp(x) - 1.0))


def _safe_pow(t, inv):
    pos = t > 0.0
    lg = jnp.log2(jnp.where(pos, t, 1.0))
    return jnp.where(pos, jnp.exp2(inv * lg), 0.0)


def _full_spec(shape):
    return pl.BlockSpec(shape, lambda i: tuple(0 for _ in shape))


def _hop_scores(en, en3, u, h, w1_ref, b1_ref, w2_ref, b2_ref, w3t_ref,
                b3_ref):
    tb = en.shape[0] // K
    u_n = _normalize_rows(u)
    a_e = jnp.dot(en, w1_ref[h, :D, :], preferred_element_type=jnp.float32)
    a_u = jnp.dot(u_n, w1_ref[h, D:, :], preferred_element_type=jnp.float32)
    b1 = b1_ref[h:h + 1, :].reshape(1, 1, D)
    a1 = _selu(a_e.reshape(K, tb, D) + a_u[None, :, :] + b1)
    a2 = _selu(jnp.dot(a1.reshape(K * tb, D), w2_ref[h],
                       preferred_element_type=jnp.float32)
               + b2_ref[h:h + 1, :])
    w3 = w3t_ref[h].reshape(1, 1, D4)
    return jnp.sum(a2.reshape(K, tb, D4) * w3, axis=-1) + b3_ref[h:h + 1, 0:1]


def _aggregate(en3, attT):
    att = jnp.transpose(attT)
    u = jnp.zeros((en3.shape[1], D), jnp.float32)
    for k in range(K):
        u = u + en3[k] * att[:, k:k + 1]
    return u


def _s1_body(e_ref, s_ref, w1_ref, b1_ref, w2_ref, b2_ref, w3t_ref, b3_ref,
             l1t_ref, l1b_ref, en_ref, s1_ref, w1o_ref, w2o_ref):
    tb = e_ref.shape[1]
    en = _normalize_rows(e_ref[...].reshape(K * tb, D))
    en3 = en.reshape(K, tb, D)
    en_ref[...] = en3.astype(jnp.bfloat16)
    for h in range(H):
        l1 = l1t_ref[h].reshape(1, 1, D)
        wT = jax.nn.sigmoid(jnp.sum(en3 * l1, axis=-1)
                            + l1b_ref[h:h + 1, 0:1]) + 1.0
        if h == 0:
            w1o_ref[...] = wT
        else:
            w2o_ref[...] = wT
    s1_ref[...] = _hop_scores(en, en3, s_ref[...], 0, w1_ref, b1_ref, w2_ref,
                              b2_ref, w3t_ref, b3_ref)


def _s1(e_kmaj, sf, att1_W, att1_b, att2_W, att2_b, att3_Wt, att3_b, lin1_Wt,
        lin1_b):
    return pl.pallas_call(
        _s1_body,
        grid=(B // _TB_S1,),
        in_specs=[
            pl.BlockSpec((K, _TB_S1, D), lambda i: (0, i, 0)),
            pl.BlockSpec((_TB_S1, D), lambda i: (i, 0)),
            _full_spec((H, 2 * D, D)),
            _full_spec((H, D)),
            _full_spec((H, D, D4)),
            _full_spec((H, D4)),
            _full_spec((H, 1, D4)),
            _full_spec((H, 1)),
            _full_spec((H, 1, D)),
            _full_spec((H, 1)),
        ],
        out_specs=[
            pl.BlockSpec((K, _TB_S1, D), lambda i: (0, i, 0)),
            pl.BlockSpec((K, _TB_S1), lambda i: (0, i)),
            pl.BlockSpec((K, _TB_S1), lambda i: (0, i)),
            pl.BlockSpec((K, _TB_S1), lambda i: (0, i)),
        ],
        out_shape=[
            jax.ShapeDtypeStruct((K, B, D), jnp.bfloat16),
            jax.ShapeDtypeStruct((K, B), jnp.float32),
            jax.ShapeDtypeStruct((K, B), jnp.float32),
            jax.ShapeDtypeStruct((K, B), jnp.float32),
        ],
    )(e_kmaj, sf, att1_W, att1_b, att2_W, att2_b, att3_Wt, att3_b, lin1_Wt,
      lin1_b)


def _entmax_body(s_ref, w_ref, out_ref):
    x = s_ref[...]
    alpha = w_ref[...]
    am1 = alpha - 1.0
    xs = x * am1
    inv = 1.0 / am1
    mx = jnp.max(xs, axis=0, keepdims=True)
    lo = mx - 1.0
    hi = mx
    for _ in range(N_ITER):
        mid = 0.5 * (lo + hi)
        f = jnp.sum(_safe_pow(jnp.maximum(xs - mid, 0.0), inv), axis=0,
                    keepdims=True) - 1.0
        ge = f >= 0.0
        lo = jnp.where(ge, mid, lo)
        hi = jnp.where(ge, hi, mid)
    tau = 0.5 * (lo + hi)
    p = _safe_pow(jnp.maximum(xs - tau, 0.0), inv)
    out_ref[...] = p / jnp.sum(p, axis=0, keepdims=True)


def _entmax_fb(sT, wT):
    return pl.pallas_call(
        _entmax_body,
        out_shape=jax.ShapeDtypeStruct((K, B), jnp.float32),
    )(sT, wT)


def _a1s2_body(en_ref, att_ref, w1_ref, b1_ref, w2_ref, b2_ref, w3t_ref,
               b3_ref, u1_ref, s2_ref):
    en3 = en_ref[...].astype(jnp.float32)
    en = en3.reshape(K * en3.shape[1], D)
    u1 = _aggregate(en3, att_ref[...])
    u1_ref[...] = u1
    s2_ref[...] = _hop_scores(en, en3, u1, 1, w1_ref, b1_ref, w2_ref, b2_ref,
                              w3t_ref, b3_ref)


def _a1s2(en, attT1, att1_W, att1_b, att2_W, att2_b, att3_Wt, att3_b):
    return pl.pallas_call(
        _a1s2_body,
        grid=(B // _TB_AG,),
        in_specs=[
            pl.BlockSpec((K, _TB_AG, D), lambda i: (0, i, 0)),
            pl.BlockSpec((K, _TB_AG), lambda i: (0, i)),
            _full_spec((H, 2 * D, D)),
            _full_spec((H, D)),
            _full_spec((H, D, D4)),
            _full_spec((H, D4)),
            _full_spec((H, 1, D4)),
            _full_spec((H, 1)),
        ],
        out_specs=[
            pl.BlockSpec((_TB_AG, D), lambda i: (i, 0)),
            pl.BlockSpec((K, _TB_AG), lambda i: (0, i)),
        ],
        out_shape=[
            jax.ShapeDtypeStruct((B, D), jnp.float32),
            jax.ShapeDtypeStruct((K, B), jnp.float32),
        ],
    )(en, attT1, att1_W, att1_b, att2_W, att2_b, att3_Wt, att3_b)


def _a2_body(en_ref, att_ref, u1_ref, nf_ref):
    u2 = _aggregate(en_ref[...].astype(jnp.float32), att_ref[...])
    nf_ref[...] = (u1_ref[...] + u2) * 0.5


def _a2(en, attT2, u1):
    return pl.pallas_call(
        _a2_body,
        grid=(B // _TB_AG,),
        in_specs=[
            pl.BlockSpec((K, _TB_AG, D), lambda i: (0, i, 0)),
            pl.BlockSpec((K, _TB_AG), lambda i: (0, i)),
            pl.BlockSpec((_TB_AG, D), lambda i: (i, 0)),
        ],
        out_specs=pl.BlockSpec((_TB_AG, D), lambda i: (i, 0)),
        out_shape=jax.ShapeDtypeStruct((B, D), jnp.float32),
    )(en, attT2, u1)


def _bmean(x):
    s = jnp.sum(x.reshape(32, B // 32, D), axis=0)
    return jnp.sum(s, axis=0, keepdims=True) * (1.0 / B)


def _head_body(nf_ref, sf_ref, inw_ref, inb_ref, outw_ref, outb_ref, gw_ref,
               gb_ref, bng_ref, bnb_ref, bn1g_ref, bn1b_ref, o_ref):
    nf = nf_ref[...]
    sf = sf_ref[...]
    mu = _bmean(nf)
    xc = nf - mu
    var = _bmean(xc * xc)
    h0 = xc / jnp.sqrt(var + 1e-5) * bng_ref[...] + bnb_ref[...]
    h1 = _selu(jnp.dot(h0, inw_ref[...], preferred_element_type=jnp.float32)
               + inb_ref[...])
    mu1 = _bmean(h1)
    xc1 = h1 - mu1
    var1 = _bmean(xc1 * xc1)
    h1n = xc1 / jnp.sqrt(var1 + 1e-5) * bn1g_ref[...] + bn1b_ref[...]
    no = jnp.dot(h1n, outw_ref[...], preferred_element_type=jnp.float32) \
        + outb_ref[...]
    z = (jnp.dot(sf, gw_ref[0:D, :], preferred_element_type=jnp.float32)
         + jnp.dot(no, gw_ref[D:2 * D, :], preferred_element_type=jnp.float32)
         + jnp.dot(sf * no, gw_ref[2 * D:, :],
                   preferred_element_type=jnp.float32)
         + gb_ref[...])
    gama = jax.nn.sigmoid(z)
    o_ref[...] = gama * sf + (1.0 - gama) * no


def _head(nf, sf, in_W, in_b, out_W, out_b, gate_W, gate_b, bn_g, bn_b,
          bn1_g, bn1_b):
    return pl.pallas_call(
        _head_body,
        out_shape=jax.ShapeDtypeStruct((B, D), jnp.float32),
    )(nf, sf, in_W, in_b, out_W, out_b, gate_W, gate_b, bn_g, bn_b,
      bn1_g, bn1_b)


def kernel(nodes, neighbors, u2e, att1_W, att1_b, att2_W, att2_b, att3_W,
           att3_b, lin1_W, lin1_b, gate_W, gate_b, bn_g, bn_b, in_W, in_b,
           bn1_g, bn1_b, out_W, out_b):
    idx = jnp.concatenate([neighbors.T.reshape(-1).astype(jnp.int32),
                           nodes.astype(jnp.int32)])
    g = _sc_gather(u2e, idx)
    e_kmaj = g[:B * K].reshape(K, B, D)
    sf = g[B * K:]
    att3_Wt = jnp.transpose(att3_W, (0, 2, 1))
    lin1_Wt = jnp.transpose(lin1_W, (0, 2, 1))
    en, sT1, wT1, wT2 = _s1(e_kmaj, sf, att1_W, att1_b, att2_W, att2_b,
                            att3_Wt, att3_b, lin1_Wt, lin1_b)
    attT1 = _entmax_fb(sT1, wT1)
    u1, sT2 = _a1s2(en, attT1, att1_W, att1_b, att2_W, att2_b, att3_Wt,
                    att3_b)
    attT2 = _entmax_fb(sT2, wT2)
    nf = _a2(en, attT2, u1)
    return _head(nf, sf, in_W, in_b.reshape(1, D), out_W, out_b.reshape(1, D),
                 gate_W, gate_b.reshape(1, D), bn_g.reshape(1, D),
                 bn_b.reshape(1, D), bn1_g.reshape(1, D), bn1_b.reshape(1, D))


# speedup vs baseline: 7.2009x; 1.0005x over previous
import jax
import jax.numpy as jnp
from jax import lax
from jax.experimental import pallas as pl
from jax.experimental.pallas import tpu as pltpu
from jax.experimental.pallas import tpu_sc as plsc

D = 128
B = 4096
K = 32
H = 2
D4 = 32
N_ITER = 30

_SELU_ALPHA = 1.6732632423543772
_SELU_SCALE = 1.0507009873554805

_NC, _NS = 2, 16
_NW = _NC * _NS
_NIDX = B * K + B
_RPW = _NIDX // _NW
_CH = 352
_NCHUNK = _RPW // _CH


def _sc_gather_body(table, idx_hbm, out_hbm, idx_v, buf0, buf1, gs0, gs1,
                    ws0, ws1):
    wid = lax.axis_index("s") * _NC + lax.axis_index("c")
    base = wid * _RPW
    pltpu.sync_copy(idx_hbm.at[pl.ds(base, _RPW)], idx_v)
    bufs = (buf0, buf1)
    gsems = (gs0, gs1)
    wsems = (ws0, ws1)
    gcp = [None, None]
    wcp = [None, None]
    for j in range(_NCHUNK):
        b = j % 2
        if wcp[b] is not None:
            wcp[b].wait()
        gcp[b] = pltpu.async_copy(
            table.at[idx_v.at[pl.ds(j * _CH, _CH)]], bufs[b], gsems[b])
        if j > 0:
            pb = (j - 1) % 2
            gcp[pb].wait()
            wcp[pb] = pltpu.async_copy(
                bufs[pb], out_hbm.at[pl.ds(base + (j - 1) * _CH, _CH)],
                wsems[pb])
    lb = (_NCHUNK - 1) % 2
    gcp[lb].wait()
    wcp[lb] = pltpu.async_copy(
        bufs[lb], out_hbm.at[pl.ds(base + (_NCHUNK - 1) * _CH, _CH)],
        wsems[lb])
    wcp[0].wait()
    wcp[1].wait()


def _sc_gather(u2e, idx):
    f = pl.kernel(
        _sc_gather_body,
        mesh=plsc.VectorSubcoreMesh(core_axis_name="c", subcore_axis_name="s"),
        out_type=jax.ShapeDtypeStruct((_NIDX, D), jnp.float32),
        scratch_types=[
            pltpu.VMEM((_RPW,), jnp.int32),
            pltpu.VMEM((_CH, D), jnp.float32),
            pltpu.VMEM((_CH, D), jnp.float32),
            pltpu.SemaphoreType.DMA,
            pltpu.SemaphoreType.DMA,
            pltpu.SemaphoreType.DMA,
            pltpu.SemaphoreType.DMA,
        ],
    )
    return f(u2e, idx)


_TB_S1 = 512
_TB_AG = 512


def _normalize_rows(x):
    n = jnp.sqrt(jnp.sum(x * x, axis=-1, keepdims=True))
    return x / jnp.maximum(n, 1e-12)


def _selu(x):
    return _SELU_SCALE * jnp.where(x > 0, x, _SELU_ALPHA * (jnp.exp(x) - 1.0))


def _safe_pow(t, inv):
    pos = t > 0.0
    lg = jnp.log2(jnp.where(pos, t, 1.0))
    return jnp.where(pos, jnp.exp2(inv * lg), 0.0)


def _full_spec(shape):
    return pl.BlockSpec(shape, lambda i: tuple(0 for _ in shape))


def _hop_scores(en, en3, u, h, w1_ref, b1_ref, w2_ref, b2_ref, w3t_ref,
                b3_ref):
    tb = en.shape[0] // K
    u_n = _normalize_rows(u)
    a_e = jnp.dot(en, w1_ref[h, :D, :], preferred_element_type=jnp.float32)
    a_u = jnp.dot(u_n, w1_ref[h, D:, :], preferred_element_type=jnp.float32)
    b1 = b1_ref[h:h + 1, :].reshape(1, 1, D)
    a1 = _selu(a_e.reshape(K, tb, D) + a_u[None, :, :] + b1)
    a2 = _selu(jnp.dot(a1.reshape(K * tb, D), w2_ref[h],
                       preferred_element_type=jnp.float32)
               + b2_ref[h:h + 1, :])
    w3 = w3t_ref[h].reshape(1, 1, D4)
    return jnp.sum(a2.reshape(K, tb, D4) * w3, axis=-1) + b3_ref[h:h + 1, 0:1]


def _aggregate(en3, attT):
    att = jnp.transpose(attT)
    u = jnp.zeros((en3.shape[1], D), jnp.float32)
    for k in range(K):
        u = u + en3[k] * att[:, k:k + 1]
    return u


def _s1_body(e_ref, s_ref, w1_ref, b1_ref, w2_ref, b2_ref, w3t_ref, b3_ref,
             l1t_ref, l1b_ref, en_ref, s1_ref, w1o_ref, w2o_ref):
    tb = e_ref.shape[1]
    en = _normalize_rows(e_ref[...].reshape(K * tb, D))
    en3 = en.reshape(K, tb, D)
    en_ref[...] = en3.astype(jnp.bfloat16)
    for h in range(H):
        l1 = l1t_ref[h].reshape(1, 1, D)
        wT = jax.nn.sigmoid(jnp.sum(en3 * l1, axis=-1)
                            + l1b_ref[h:h + 1, 0:1]) + 1.0
        if h == 0:
            w1o_ref[...] = wT
        else:
            w2o_ref[...] = wT
    s1_ref[...] = _hop_scores(en, en3, s_ref[...], 0, w1_ref, b1_ref, w2_ref,
                              b2_ref, w3t_ref, b3_ref)


def _s1(e_kmaj, sf, att1_W, att1_b, att2_W, att2_b, att3_Wt, att3_b, lin1_Wt,
        lin1_b):
    return pl.pallas_call(
        _s1_body,
        grid=(B // _TB_S1,),
        in_specs=[
            pl.BlockSpec((K, _TB_S1, D), lambda i: (0, i, 0)),
            pl.BlockSpec((_TB_S1, D), lambda i: (i, 0)),
            _full_spec((H, 2 * D, D)),
            _full_spec((H, D)),
            _full_spec((H, D, D4)),
            _full_spec((H, D4)),
            _full_spec((H, 1, D4)),
            _full_spec((H, 1)),
            _full_spec((H, 1, D)),
            _full_spec((H, 1)),
        ],
        out_specs=[
            pl.BlockSpec((K, _TB_S1, D), lambda i: (0, i, 0)),
            pl.BlockSpec((K, _TB_S1), lambda i: (0, i)),
            pl.BlockSpec((K, _TB_S1), lambda i: (0, i)),
            pl.BlockSpec((K, _TB_S1), lambda i: (0, i)),
        ],
        out_shape=[
            jax.ShapeDtypeStruct((K, B, D), jnp.bfloat16),
            jax.ShapeDtypeStruct((K, B), jnp.float32),
            jax.ShapeDtypeStruct((K, B), jnp.float32),
            jax.ShapeDtypeStruct((K, B), jnp.float32),
        ],
    )(e_kmaj, sf, att1_W, att1_b, att2_W, att2_b, att3_Wt, att3_b, lin1_Wt,
      lin1_b)


def _entmax_body(s_ref, w_ref, out_ref):
    x = s_ref[...]
    alpha = w_ref[...]
    am1 = alpha - 1.0
    xs = x * am1
    inv = 1.0 / am1
    mx = jnp.max(xs, axis=0, keepdims=True)
    lo = mx - 1.0
    hi = mx
    for _ in range(N_ITER):
        mid = 0.5 * (lo + hi)
        f = jnp.sum(_safe_pow(jnp.maximum(xs - mid, 0.0), inv), axis=0,
                    keepdims=True) - 1.0
        ge = f >= 0.0
        lo = jnp.where(ge, mid, lo)
        hi = jnp.where(ge, hi, mid)
    tau = 0.5 * (lo + hi)
    p = _safe_pow(jnp.maximum(xs - tau, 0.0), inv)
    out_ref[...] = p / jnp.sum(p, axis=0, keepdims=True)


def _entmax_fb(sT, wT):
    return pl.pallas_call(
        _entmax_body,
        out_shape=jax.ShapeDtypeStruct((K, B), jnp.float32),
    )(sT, wT)


def _a1s2_body(en_ref, att_ref, w1_ref, b1_ref, w2_ref, b2_ref, w3t_ref,
               b3_ref, u1_ref, s2_ref):
    en3 = en_ref[...].astype(jnp.float32)
    en = en3.reshape(K * en3.shape[1], D)
    u1 = _aggregate(en3, att_ref[...])
    u1_ref[...] = u1
    s2_ref[...] = _hop_scores(en, en3, u1, 1, w1_ref, b1_ref, w2_ref, b2_ref,
                              w3t_ref, b3_ref)


def _a1s2(en, attT1, att1_W, att1_b, att2_W, att2_b, att3_Wt, att3_b):
    return pl.pallas_call(
        _a1s2_body,
        grid=(B // _TB_AG,),
        in_specs=[
            pl.BlockSpec((K, _TB_AG, D), lambda i: (0, i, 0)),
            pl.BlockSpec((K, _TB_AG), lambda i: (0, i)),
            _full_spec((H, 2 * D, D)),
            _full_spec((H, D)),
            _full_spec((H, D, D4)),
            _full_spec((H, D4)),
            _full_spec((H, 1, D4)),
            _full_spec((H, 1)),
        ],
        out_specs=[
            pl.BlockSpec((_TB_AG, D), lambda i: (i, 0)),
            pl.BlockSpec((K, _TB_AG), lambda i: (0, i)),
        ],
        out_shape=[
            jax.ShapeDtypeStruct((B, D), jnp.float32),
            jax.ShapeDtypeStruct((K, B), jnp.float32),
        ],
    )(en, attT1, att1_W, att1_b, att2_W, att2_b, att3_Wt, att3_b)


def _a2_body(en_ref, att_ref, u1_ref, nf_ref):
    u2 = _aggregate(en_ref[...].astype(jnp.float32), att_ref[...])
    nf_ref[...] = (u1_ref[...] + u2) * 0.5


def _a2(en, attT2, u1):
    return pl.pallas_call(
        _a2_body,
        grid=(B // _TB_AG,),
        in_specs=[
            pl.BlockSpec((K, _TB_AG, D), lambda i: (0, i, 0)),
            pl.BlockSpec((K, _TB_AG), lambda i: (0, i)),
            pl.BlockSpec((_TB_AG, D), lambda i: (i, 0)),
        ],
        out_specs=pl.BlockSpec((_TB_AG, D), lambda i: (i, 0)),
        out_shape=jax.ShapeDtypeStruct((B, D), jnp.float32),
    )(en, attT2, u1)


def _bmean(x):
    s = jnp.sum(x.reshape(32, B // 32, D), axis=0)
    return jnp.sum(s, axis=0, keepdims=True) * (1.0 / B)


def _head_body(nf_ref, sf_ref, inw_ref, inb_ref, outw_ref, outb_ref, gw_ref,
               gb_ref, bng_ref, bnb_ref, bn1g_ref, bn1b_ref, o_ref):
    nf = nf_ref[...]
    sf = sf_ref[...]
    mu = _bmean(nf)
    xc = nf - mu
    var = _bmean(xc * xc)
    h0 = xc / jnp.sqrt(var + 1e-5) * bng_ref[...] + bnb_ref[...]
    h1 = _selu(jnp.dot(h0, inw_ref[...], preferred_element_type=jnp.float32)
               + inb_ref[...])
    mu1 = _bmean(h1)
    xc1 = h1 - mu1
    var1 = _bmean(xc1 * xc1)
    h1n = xc1 / jnp.sqrt(var1 + 1e-5) * bn1g_ref[...] + bn1b_ref[...]
    no = jnp.dot(h1n, outw_ref[...], preferred_element_type=jnp.float32) \
        + outb_ref[...]
    z = (jnp.dot(sf, gw_ref[0:D, :], preferred_element_type=jnp.float32)
         + jnp.dot(no, gw_ref[D:2 * D, :], preferred_element_type=jnp.float32)
         + jnp.dot(sf * no, gw_ref[2 * D:, :],
                   preferred_element_type=jnp.float32)
         + gb_ref[...])
    gama = jax.nn.sigmoid(z)
    o_ref[...] = gama * sf + (1.0 - gama) * no


def _head(nf, sf, in_W, in_b, out_W, out_b, gate_W, gate_b, bn_g, bn_b,
          bn1_g, bn1_b):
    return pl.pallas_call(
        _head_body,
        out_shape=jax.ShapeDtypeStruct((B, D), jnp.float32),
    )(nf, sf, in_W, in_b, out_W, out_b, gate_W, gate_b, bn_g, bn_b,
      bn1_g, bn1_b)


def kernel(nodes, neighbors, u2e, att1_W, att1_b, att2_W, att2_b, att3_W,
           att3_b, lin1_W, lin1_b, gate_W, gate_b, bn_g, bn_b, in_W, in_b,
           bn1_g, bn1_b, out_W, out_b):
    idx = jnp.concatenate([neighbors.T.reshape(-1).astype(jnp.int32),
                           nodes.astype(jnp.int32)])
    g = _sc_gather(u2e, idx)
    e_kmaj = g[:B * K].reshape(K, B, D)
    sf = g[B * K:]
    att3_Wt = jnp.transpose(att3_W, (0, 2, 1))
    lin1_Wt = jnp.transpose(lin1_W, (0, 2, 1))
    en, sT1, wT1, wT2 = _s1(e_kmaj, sf, att1_W, att1_b, att2_W, att2_b,
                            att3_Wt, att3_b, lin1_Wt, lin1_b)
    attT1 = _entmax_fb(sT1, wT1)
    u1, sT2 = _a1s2(en, attT1, att1_W, att1_b, att2_W, att2_b, att3_Wt,
                    att3_b)
    attT2 = _entmax_fb(sT2, wT2)
    nf = _a2(en, attT2, u1)
    return _head(nf, sf, in_W, in_b.reshape(1, D), out_W, out_b.reshape(1, D),
                 gate_W, gate_b.reshape(1, D), bn_g.reshape(1, D),
                 bn_b.reshape(1, D), bn1_g.reshape(1, D), bn1_b.reshape(1, D))
